# Initial kernel scaffold; baseline (speedup 1.0000x reference)
import jax
import jax.numpy as jnp
from jax.experimental import pallas as pl


def kernel(x, node_topo_features, noise, w_gate, w_noise, W1, b1, W2, b2):
    raise NotImplementedError("write your pallas kernel here")


# speedup vs baseline: 1.2267x; 1.2267x over previous
import functools

import jax
import jax.numpy as jnp
from jax import lax
from jax.experimental import pallas as pl

N_TOKENS = 8192
D_MODEL = 768
EXTRA_DIM = 24
HIDDEN = 256
N_EXPERTS = 8
TOP_K = 2
LOSS_COEF = 0.01

TILE = 512
NTILES = N_TOKENS // TILE


def _cv_sq(v):
    eps = 1e-10
    m = jnp.mean(v)
    var = jnp.mean((v - m) ** 2)
    return var / (m * m + eps)


def _moe_body(x_ref, topo_ref, noise_ref, wcat_ref, w1x_ref, w1t_ref,
              b1f_ref, w2f_ref, b2f_ref,
              y_ref, imp_ref, load_ref, aux_ref):
    i = pl.program_id(0)
    x = x_ref[...]
    logits16 = jnp.dot(x, wcat_ref[...], preferred_element_type=jnp.float32)
    clean = logits16[:, :N_EXPERTS]
    raw = logits16[:, N_EXPERTS:2 * N_EXPERTS]
    sp = jnp.maximum(raw, 0.0) + jnp.log1p(jnp.exp(-jnp.abs(raw)))
    noisy = clean + noise_ref[...] * (sp + 1e-2)

    eio = lax.broadcasted_iota(jnp.int32, (TILE, N_EXPERTS), 1)
    v1 = jnp.max(noisy, axis=1, keepdims=True)
    idx1 = jnp.min(jnp.where(noisy >= v1, eio, N_EXPERTS), axis=1,
                   keepdims=True)
    sel1 = eio == idx1
    noisy2 = jnp.where(sel1, -jnp.inf, noisy)
    v2 = jnp.max(noisy2, axis=1, keepdims=True)
    idx2 = jnp.min(jnp.where(noisy2 >= v2, eio, N_EXPERTS), axis=1,
                   keepdims=True)
    sel2 = eio == idx2
    t = jnp.exp(v2 - v1)
    g1 = 1.0 / (1.0 + t)
    g2 = t / (1.0 + t)
    gates = jnp.where(sel1, g1, 0.0) + jnp.where(sel2, g2, 0.0)

    imp_part = jnp.sum(gates, axis=0)[None, :]
    load_part = jnp.sum((gates > 0.0).astype(jnp.float32), axis=0)[None, :]

    @pl.when(i == 0)
    def _():
        imp_ref[...] = imp_part
        load_ref[...] = load_part

    @pl.when(i > 0)
    def _():
        imp_ref[...] += imp_part
        load_ref[...] += load_part

    h = (jnp.dot(x, w1x_ref[...], preferred_element_type=jnp.float32)
         + jnp.dot(topo_ref[...], w1t_ref[...],
                   preferred_element_type=jnp.float32)
         + b1f_ref[...])
    h = jnp.maximum(h, 0.0)
    sall = h * w2f_ref[...]
    yt = jnp.zeros((TILE,), jnp.float32)
    for e in range(N_EXPERTS):
        s_e = jnp.sum(sall[:, e * HIDDEN:(e + 1) * HIDDEN], axis=1)
        yt += gates[:, e] * (s_e + b2f_ref[0, e])
    y_ref[...] = yt

    @pl.when(i == NTILES - 1)
    def _():
        aux = LOSS_COEF * (_cv_sq(imp_ref[0, :]) + _cv_sq(load_ref[0, :]))
        aux_ref[...] = aux.reshape(1, 1)


@functools.partial(jax.jit, static_argnames=("interpret",))
def kernel(x, node_topo_features, noise, w_gate, w_noise, W1, b1, W2, b2,
           interpret=False):
    wcat = jnp.concatenate([w_gate, w_noise], axis=1)
    w1r = jnp.transpose(W1, (1, 0, 2)).reshape(D_MODEL + EXTRA_DIM,
                                               N_EXPERTS * HIDDEN)
    w1x = w1r[:D_MODEL]
    w1t = w1r[D_MODEL:]
    b1f = b1.reshape(1, N_EXPERTS * HIDDEN)
    w2f = W2[:, :, 0].reshape(1, N_EXPERTS * HIDDEN)
    b2f = b2.reshape(1, N_EXPERTS)

    grid = (NTILES,)
    y2d, _imp, _load, aux = pl.pallas_call(
        _moe_body,
        grid=grid,
        in_specs=[
            pl.BlockSpec((TILE, D_MODEL), lambda i: (i, 0)),
            pl.BlockSpec((TILE, EXTRA_DIM), lambda i: (i, 0)),
            pl.BlockSpec((TILE, N_EXPERTS), lambda i: (i, 0)),
            pl.BlockSpec((D_MODEL, 2 * N_EXPERTS), lambda i: (0, 0)),
            pl.BlockSpec((D_MODEL, N_EXPERTS * HIDDEN), lambda i: (0, 0)),
            pl.BlockSpec((EXTRA_DIM, N_EXPERTS * HIDDEN), lambda i: (0, 0)),
            pl.BlockSpec((1, N_EXPERTS * HIDDEN), lambda i: (0, 0)),
            pl.BlockSpec((1, N_EXPERTS * HIDDEN), lambda i: (0, 0)),
            pl.BlockSpec((1, N_EXPERTS), lambda i: (0, 0)),
        ],
        out_specs=[
            pl.BlockSpec((TILE,), lambda i: (i,)),
            pl.BlockSpec((1, N_EXPERTS), lambda i: (0, 0)),
            pl.BlockSpec((1, N_EXPERTS), lambda i: (0, 0)),
            pl.BlockSpec((1, 1), lambda i: (0, 0)),
        ],
        out_shape=[
            jax.ShapeDtypeStruct((N_TOKENS,), jnp.float32),
            jax.ShapeDtypeStruct((1, N_EXPERTS), jnp.float32),
            jax.ShapeDtypeStruct((1, N_EXPERTS), jnp.float32),
            jax.ShapeDtypeStruct((1, 1), jnp.float32),
        ],
        interpret=interpret,
    )(x, node_topo_features, noise, wcat, w1x, w1t, b1f, w2f, b2f)
    return y2d, aux[0, 0]


# speedup vs baseline: 1.5554x; 1.2679x over previous
import functools

import jax
import jax.numpy as jnp
from jax import lax
from jax.experimental import pallas as pl

N_TOKENS = 8192
D_MODEL = 768
EXTRA_DIM = 24
HIDDEN = 256
N_EXPERTS = 8
TOP_K = 2
LOSS_COEF = 0.01

TILE = 512
NTILES = N_TOKENS // TILE


def _cv_sq(v):
    eps = 1e-10
    m = jnp.mean(v)
    var = jnp.mean((v - m) ** 2)
    return var / (m * m + eps)


def _moe_body(x_ref, topo_ref, noise_ref, wcat_ref, w1x_ref, w1t_ref,
              b1f_ref, w2f_ref, b2f_ref,
              y_ref, imp_ref, load_ref, aux_ref):
    i = pl.program_id(0)
    x = x_ref[...]
    logits16 = jnp.dot(x, wcat_ref[...], preferred_element_type=jnp.float32)
    clean = logits16[:, :N_EXPERTS]
    raw = logits16[:, N_EXPERTS:2 * N_EXPERTS]
    sp = jnp.maximum(raw, 0.0) + jnp.log1p(jnp.exp(-jnp.abs(raw)))
    noisy = clean + noise_ref[...] * (sp + 1e-2)

    eio = lax.broadcasted_iota(jnp.int32, (TILE, N_EXPERTS), 1)
    v1 = jnp.max(noisy, axis=1, keepdims=True)
    idx1 = jnp.min(jnp.where(noisy >= v1, eio, N_EXPERTS), axis=1,
                   keepdims=True)
    sel1 = eio == idx1
    noisy2 = jnp.where(sel1, -jnp.inf, noisy)
    v2 = jnp.max(noisy2, axis=1, keepdims=True)
    idx2 = jnp.min(jnp.where(noisy2 >= v2, eio, N_EXPERTS), axis=1,
                   keepdims=True)
    sel2 = eio == idx2
    t = jnp.exp(v2 - v1)
    g1 = 1.0 / (1.0 + t)
    g2 = t / (1.0 + t)
    gates = jnp.where(sel1, g1, 0.0) + jnp.where(sel2, g2, 0.0)

    imp_part = jnp.sum(gates, axis=0)[None, :]
    load_part = jnp.sum((gates > 0.0).astype(jnp.float32), axis=0)[None, :]

    @pl.when(i == 0)
    def _():
        imp_ref[...] = imp_part
        load_ref[...] = load_part

    @pl.when(i > 0)
    def _():
        imp_ref[...] += imp_part
        load_ref[...] += load_part

    h = (jnp.dot(x.astype(jnp.bfloat16), w1x_ref[...],
                 preferred_element_type=jnp.float32)
         + jnp.dot(topo_ref[...].astype(jnp.bfloat16), w1t_ref[...],
                   preferred_element_type=jnp.float32)
         + b1f_ref[...])
    h = jnp.maximum(h, 0.0)
    sall = h * w2f_ref[...]
    yt = jnp.zeros((TILE,), jnp.float32)
    for e in range(N_EXPERTS):
        s_e = jnp.sum(sall[:, e * HIDDEN:(e + 1) * HIDDEN], axis=1)
        yt += gates[:, e] * (s_e + b2f_ref[0, e])
    y_ref[...] = yt

    @pl.when(i == NTILES - 1)
    def _():
        aux = LOSS_COEF * (_cv_sq(imp_ref[0, :]) + _cv_sq(load_ref[0, :]))
        aux_ref[...] = aux.reshape(1, 1)


@functools.partial(jax.jit, static_argnames=("interpret",))
def kernel(x, node_topo_features, noise, w_gate, w_noise, W1, b1, W2, b2,
           interpret=False):
    wcat = jnp.concatenate([w_gate, w_noise], axis=1)
    w1r = jnp.transpose(W1, (1, 0, 2)).reshape(D_MODEL + EXTRA_DIM,
                                               N_EXPERTS * HIDDEN)
    w1x = w1r[:D_MODEL].astype(jnp.bfloat16)
    w1t = w1r[D_MODEL:].astype(jnp.bfloat16)
    b1f = b1.reshape(1, N_EXPERTS * HIDDEN)
    w2f = W2[:, :, 0].reshape(1, N_EXPERTS * HIDDEN)
    b2f = b2.reshape(1, N_EXPERTS)

    grid = (NTILES,)
    y2d, _imp, _load, aux = pl.pallas_call(
        _moe_body,
        grid=grid,
        in_specs=[
            pl.BlockSpec((TILE, D_MODEL), lambda i: (i, 0)),
            pl.BlockSpec((TILE, EXTRA_DIM), lambda i: (i, 0)),
            pl.BlockSpec((TILE, N_EXPERTS), lambda i: (i, 0)),
            pl.BlockSpec((D_MODEL, 2 * N_EXPERTS), lambda i: (0, 0)),
            pl.BlockSpec((D_MODEL, N_EXPERTS * HIDDEN), lambda i: (0, 0)),
            pl.BlockSpec((EXTRA_DIM, N_EXPERTS * HIDDEN), lambda i: (0, 0)),
            pl.BlockSpec((1, N_EXPERTS * HIDDEN), lambda i: (0, 0)),
            pl.BlockSpec((1, N_EXPERTS * HIDDEN), lambda i: (0, 0)),
            pl.BlockSpec((1, N_EXPERTS), lambda i: (0, 0)),
        ],
        out_specs=[
            pl.BlockSpec((TILE,), lambda i: (i,)),
            pl.BlockSpec((1, N_EXPERTS), lambda i: (0, 0)),
            pl.BlockSpec((1, N_EXPERTS), lambda i: (0, 0)),
            pl.BlockSpec((1, 1), lambda i: (0, 0)),
        ],
        out_shape=[
            jax.ShapeDtypeStruct((N_TOKENS,), jnp.float32),
            jax.ShapeDtypeStruct((1, N_EXPERTS), jnp.float32),
            jax.ShapeDtypeStruct((1, N_EXPERTS), jnp.float32),
            jax.ShapeDtypeStruct((1, 1), jnp.float32),
        ],
        interpret=interpret,
    )(x, node_topo_features, noise, wcat, w1x, w1t, b1f, w2f, b2f)
    return y2d, aux[0, 0]


# speedup vs baseline: 2.5147x; 1.6168x over previous
import functools

import jax
import jax.numpy as jnp
from jax import lax
from jax.experimental import pallas as pl

N_TOKENS = 8192
D_MODEL = 768
EXTRA_DIM = 24
HIDDEN = 256
N_EXPERTS = 8
TOP_K = 2
LOSS_COEF = 0.01

TILE = 512
NTILES = N_TOKENS // TILE


def _cv_sq(v):
    eps = 1e-10
    m = jnp.mean(v)
    var = jnp.mean((v - m) ** 2)
    return var / (m * m + eps)


def _nt_dot(a, b):
    return lax.dot_general(a, b, (((1,), (1,)), ((), ())),
                           preferred_element_type=jnp.float32)


def _moe_body(x_ref, topo_ref, noiset_ref, wcatt_ref, w1x_ref, w1t_ref,
              b1f_ref, w2blkt_ref, b2_ref,
              y_ref, imp_ref, load_ref, aux_ref):
    i = pl.program_id(0)
    x = x_ref[...]
    logt = _nt_dot(wcatt_ref[...], x)
    clean = logt[:N_EXPERTS, :]
    raw = logt[N_EXPERTS:, :]
    sp = jnp.maximum(raw, 0.0) + jnp.log1p(jnp.exp(-jnp.abs(raw)))
    noisy = clean + noiset_ref[...] * (sp + 1e-2)

    sio = lax.broadcasted_iota(jnp.int32, (N_EXPERTS, TILE), 0)
    v1 = jnp.max(noisy, axis=0, keepdims=True)
    idx1 = jnp.min(jnp.where(noisy >= v1, sio, N_EXPERTS), axis=0,
                   keepdims=True)
    sel1 = sio == idx1
    noisy2 = jnp.where(sel1, -jnp.inf, noisy)
    v2 = jnp.max(noisy2, axis=0, keepdims=True)
    idx2 = jnp.min(jnp.where(noisy2 >= v2, sio, N_EXPERTS), axis=0,
                   keepdims=True)
    sel2 = sio == idx2
    t = jnp.exp(v2 - v1)
    g1 = 1.0 / (1.0 + t)
    g2 = t / (1.0 + t)
    gatest = jnp.where(sel1, g1, 0.0) + jnp.where(sel2, g2, 0.0)

    imp_part = jnp.sum(gatest, axis=1, keepdims=True)
    load_part = jnp.sum((gatest > 0.0).astype(jnp.float32), axis=1,
                        keepdims=True)

    @pl.when(i == 0)
    def _():
        imp_ref[...] = imp_part
        load_ref[...] = load_part

    @pl.when(i > 0)
    def _():
        imp_ref[...] += imp_part
        load_ref[...] += load_part

    h = (jnp.dot(x.astype(jnp.bfloat16), w1x_ref[...],
                 preferred_element_type=jnp.float32)
         + jnp.dot(topo_ref[...], w1t_ref[...],
                   preferred_element_type=jnp.float32)
         + b1f_ref[...])
    h = jnp.maximum(h, 0.0).astype(jnp.bfloat16)
    svect = _nt_dot(w2blkt_ref[...], h)
    yt = jnp.sum(gatest * (svect + b2_ref[...]), axis=0)
    y_ref[...] = yt

    @pl.when(i == NTILES - 1)
    def _():
        aux = LOSS_COEF * (_cv_sq(imp_ref[...]) + _cv_sq(load_ref[...]))
        aux_ref[...] = aux.reshape(1, 1)


@functools.partial(jax.jit, static_argnames=("interpret",))
def kernel(x, node_topo_features, noise, w_gate, w_noise, W1, b1, W2, b2,
           interpret=False):
    wcatt = jnp.concatenate([w_gate, w_noise], axis=1).T
    noiset = noise.T
    w1r = jnp.transpose(W1, (1, 0, 2)).reshape(D_MODEL + EXTRA_DIM,
                                               N_EXPERTS * HIDDEN)
    w1x = w1r[:D_MODEL].astype(jnp.bfloat16)
    w1t = w1r[D_MODEL:].astype(jnp.bfloat16)
    topo_bf = node_topo_features.astype(jnp.bfloat16)
    b1f = b1.reshape(1, N_EXPERTS * HIDDEN)
    eye = jnp.eye(N_EXPERTS, dtype=jnp.float32)
    w2blkt = (eye[:, :, None] * W2[None, :, :, 0]).reshape(
        N_EXPERTS, N_EXPERTS * HIDDEN).astype(jnp.bfloat16)
    b2c = b2

    grid = (NTILES,)
    y, _imp, _load, aux = pl.pallas_call(
        _moe_body,
        grid=grid,
        in_specs=[
            pl.BlockSpec((TILE, D_MODEL), lambda i: (i, 0)),
            pl.BlockSpec((TILE, EXTRA_DIM), lambda i: (i, 0)),
            pl.BlockSpec((N_EXPERTS, TILE), lambda i: (0, i)),
            pl.BlockSpec((2 * N_EXPERTS, D_MODEL), lambda i: (0, 0)),
            pl.BlockSpec((D_MODEL, N_EXPERTS * HIDDEN), lambda i: (0, 0)),
            pl.BlockSpec((EXTRA_DIM, N_EXPERTS * HIDDEN), lambda i: (0, 0)),
            pl.BlockSpec((1, N_EXPERTS * HIDDEN), lambda i: (0, 0)),
            pl.BlockSpec((N_EXPERTS, N_EXPERTS * HIDDEN), lambda i: (0, 0)),
            pl.BlockSpec((N_EXPERTS, 1), lambda i: (0, 0)),
        ],
        out_specs=[
            pl.BlockSpec((TILE,), lambda i: (i,)),
            pl.BlockSpec((N_EXPERTS, 1), lambda i: (0, 0)),
            pl.BlockSpec((N_EXPERTS, 1), lambda i: (0, 0)),
            pl.BlockSpec((1, 1), lambda i: (0, 0)),
        ],
        out_shape=[
            jax.ShapeDtypeStruct((N_TOKENS,), jnp.float32),
            jax.ShapeDtypeStruct((N_EXPERTS, 1), jnp.float32),
            jax.ShapeDtypeStruct((N_EXPERTS, 1), jnp.float32),
            jax.ShapeDtypeStruct((1, 1), jnp.float32),
        ],
        interpret=interpret,
    )(x, topo_bf, noiset, wcatt, w1x, w1t, b1f, w2blkt, b2c)
    return y, aux[0, 0]


# speedup vs baseline: 2.6290x; 1.0454x over previous
import functools

import jax
import jax.numpy as jnp
from jax import lax
from jax.experimental import pallas as pl

N_TOKENS = 8192
D_MODEL = 768
EXTRA_DIM = 24
HIDDEN = 256
N_EXPERTS = 8
TOP_K = 2
LOSS_COEF = 0.01

TILE = 1024
NTILES = N_TOKENS // TILE


def _cv_sq(v):
    eps = 1e-10
    m = jnp.mean(v)
    var = jnp.mean((v - m) ** 2)
    return var / (m * m + eps)


def _nt_dot(a, b):
    return lax.dot_general(a, b, (((1,), (1,)), ((), ())),
                           preferred_element_type=jnp.float32)


def _moe_body(x_ref, topo_ref, noiset_ref, wcatt_ref, w1x_ref, w1t_ref,
              b1f_ref, w2blkt_ref, b2_ref,
              y_ref, imp_ref, load_ref, aux_ref):
    i = pl.program_id(0)
    x = x_ref[...]
    logt = _nt_dot(wcatt_ref[...], x)
    clean = logt[:N_EXPERTS, :]
    raw = logt[N_EXPERTS:, :]
    sp = jnp.maximum(raw, 0.0) + jnp.log1p(jnp.exp(-jnp.abs(raw)))
    noisy = clean + noiset_ref[...] * (sp + 1e-2)

    sio = lax.broadcasted_iota(jnp.int32, (N_EXPERTS, TILE), 0)
    v1 = jnp.max(noisy, axis=0, keepdims=True)
    idx1 = jnp.min(jnp.where(noisy >= v1, sio, N_EXPERTS), axis=0,
                   keepdims=True)
    sel1 = sio == idx1
    noisy2 = jnp.where(sel1, -jnp.inf, noisy)
    v2 = jnp.max(noisy2, axis=0, keepdims=True)
    idx2 = jnp.min(jnp.where(noisy2 >= v2, sio, N_EXPERTS), axis=0,
                   keepdims=True)
    sel2 = sio == idx2
    t = jnp.exp(v2 - v1)
    g1 = 1.0 / (1.0 + t)
    g2 = t / (1.0 + t)
    gatest = jnp.where(sel1, g1, 0.0) + jnp.where(sel2, g2, 0.0)

    imp_part = jnp.sum(gatest, axis=1, keepdims=True)
    load_part = jnp.sum((gatest > 0.0).astype(jnp.float32), axis=1,
                        keepdims=True)

    @pl.when(i == 0)
    def _():
        imp_ref[...] = imp_part
        load_ref[...] = load_part

    @pl.when(i > 0)
    def _():
        imp_ref[...] += imp_part
        load_ref[...] += load_part

    h = (jnp.dot(x.astype(jnp.bfloat16), w1x_ref[...],
                 preferred_element_type=jnp.float32)
         + jnp.dot(topo_ref[...], w1t_ref[...],
                   preferred_element_type=jnp.float32)
         + b1f_ref[...])
    h = jnp.maximum(h, 0.0).astype(jnp.bfloat16)
    svect = _nt_dot(w2blkt_ref[...], h)
    yt = jnp.sum(gatest * (svect + b2_ref[...]), axis=0)
    y_ref[...] = yt

    @pl.when(i == NTILES - 1)
    def _():
        aux = LOSS_COEF * (_cv_sq(imp_ref[...]) + _cv_sq(load_ref[...]))
        aux_ref[...] = aux.reshape(1, 1)


@functools.partial(jax.jit, static_argnames=("interpret",))
def kernel(x, node_topo_features, noise, w_gate, w_noise, W1, b1, W2, b2,
           interpret=False):
    wcatt = jnp.concatenate([w_gate, w_noise], axis=1).T
    noiset = noise.T
    w1r = jnp.transpose(W1, (1, 0, 2)).reshape(D_MODEL + EXTRA_DIM,
                                               N_EXPERTS * HIDDEN)
    w1x = w1r[:D_MODEL].astype(jnp.bfloat16)
    w1t = w1r[D_MODEL:].astype(jnp.bfloat16)
    topo_bf = node_topo_features.astype(jnp.bfloat16)
    b1f = b1.reshape(1, N_EXPERTS * HIDDEN)
    eye = jnp.eye(N_EXPERTS, dtype=jnp.float32)
    w2blkt = (eye[:, :, None] * W2[None, :, :, 0]).reshape(
        N_EXPERTS, N_EXPERTS * HIDDEN).astype(jnp.bfloat16)
    b2c = b2

    grid = (NTILES,)
    y, _imp, _load, aux = pl.pallas_call(
        _moe_body,
        grid=grid,
        in_specs=[
            pl.BlockSpec((TILE, D_MODEL), lambda i: (i, 0)),
            pl.BlockSpec((TILE, EXTRA_DIM), lambda i: (i, 0)),
            pl.BlockSpec((N_EXPERTS, TILE), lambda i: (0, i)),
            pl.BlockSpec((2 * N_EXPERTS, D_MODEL), lambda i: (0, 0)),
            pl.BlockSpec((D_MODEL, N_EXPERTS * HIDDEN), lambda i: (0, 0)),
            pl.BlockSpec((EXTRA_DIM, N_EXPERTS * HIDDEN), lambda i: (0, 0)),
            pl.BlockSpec((1, N_EXPERTS * HIDDEN), lambda i: (0, 0)),
            pl.BlockSpec((N_EXPERTS, N_EXPERTS * HIDDEN), lambda i: (0, 0)),
            pl.BlockSpec((N_EXPERTS, 1), lambda i: (0, 0)),
        ],
        out_specs=[
            pl.BlockSpec((TILE,), lambda i: (i,)),
            pl.BlockSpec((N_EXPERTS, 1), lambda i: (0, 0)),
            pl.BlockSpec((N_EXPERTS, 1), lambda i: (0, 0)),
            pl.BlockSpec((1, 1), lambda i: (0, 0)),
        ],
        out_shape=[
            jax.ShapeDtypeStruct((N_TOKENS,), jnp.float32),
            jax.ShapeDtypeStruct((N_EXPERTS, 1), jnp.float32),
            jax.ShapeDtypeStruct((N_EXPERTS, 1), jnp.float32),
            jax.ShapeDtypeStruct((1, 1), jnp.float32),
        ],
        interpret=interpret,
    )(x, topo_bf, noiset, wcatt, w1x, w1t, b1f, w2blkt, b2c)
    return y, aux[0, 0]
